# baseline (speedup 1.0000x reference)
import functools

import jax
import jax.numpy as jnp
from jax import lax
from jax.experimental import pallas as pl
from jax.experimental.pallas import tpu as pltpu
from jax.experimental.pallas import tpu_sc as plsc

VOCAB_SIZE = 1000000
EMBEDDING_DIM = 16
NUM_NS = 4
BATCH = 16384

_NC = 2
_NS = 16
_NW = _NC * _NS
_LANES = 16
_D = EMBEDDING_DIM

_B_PER_W = BATCH // _NW
_J_PER_W = _B_PER_W * (NUM_NS + 1)
_ICHUNK = 128
_TCH = _B_PER_W // _ICHUNK
_CCH = _J_PER_W // _ICHUNK


def _sc_kernel(tgt_idx_hbm, ctx_idx_hbm, ttabT_hbm, ctabT_hbm, out_hbm,
               tgt_idx_v, ctx_idx_v, tgt_vals_v, ctx_vals_v, out_v, sem):
    wid = lax.axis_index("s") * _NC + lax.axis_index("c")

    pltpu.sync_copy(tgt_idx_hbm.at[wid], tgt_idx_v)
    pltpu.sync_copy(ctx_idx_hbm.at[wid], ctx_idx_v)

    tcopies = []
    for d in range(_D):
        for c in range(_TCH):
            tcopies.append(pltpu.async_copy(
                ttabT_hbm.at[d].at[tgt_idx_v.at[c]],
                tgt_vals_v.at[d, c], sem))
    for h in tcopies:
        h.wait()

    def gather_chunk(c, carry):
        hs = []
        for d in range(_D):
            hs.append(pltpu.async_copy(
                ctabT_hbm.at[d].at[ctx_idx_v.at[c]],
                ctx_vals_v.at[d, c], sem))
        for h in hs:
            h.wait()
        return carry

    lax.fori_loop(0, _CCH, gather_chunk, 0)

    lanes = lax.iota(jnp.int32, _LANES)

    def block(k, carry):
        j = k * _LANES + lanes
        b = lax.div(j, NUM_NS + 1)
        brow = lax.shift_right_logical(b, 7)
        blane = lax.bitwise_and(b, _ICHUNK - 1)
        crow = lax.shift_right_logical(k, 3)
        coff = lax.bitwise_and(k, 7) * _LANES
        acc = jnp.zeros((_LANES,), jnp.float32)
        for d in range(_D):
            cv = ctx_vals_v[d, crow, pl.ds(coff, _LANES)]
            tv = plsc.load_gather(
                tgt_vals_v, [jnp.full((_LANES,), d, jnp.int32), brow, blane])
            acc = acc + cv * tv
        out_v[pl.ds(k * _LANES, _LANES)] = acc
        return carry

    lax.fori_loop(0, _J_PER_W // _LANES, block, 0)

    pltpu.sync_copy(out_v, out_hbm.at[wid])


@jax.jit
def kernel(target, context, target_table, context_table):
    tgt_idx = target.reshape(_NW, _TCH, _ICHUNK)
    ctx_idx = context.reshape(_NW, _CCH, _ICHUNK)

    run = pl.kernel(
        _sc_kernel,
        out_type=jax.ShapeDtypeStruct((_NW, _J_PER_W), jnp.float32),
        mesh=plsc.VectorSubcoreMesh(core_axis_name="c", subcore_axis_name="s"),
        compiler_params=pltpu.CompilerParams(
            needs_layout_passes=False, use_tc_tiling_on_sc=False),
        scratch_types=[
            pltpu.VMEM((_TCH, _ICHUNK), jnp.int32),
            pltpu.VMEM((_CCH, _ICHUNK), jnp.int32),
            pltpu.VMEM((_D, _TCH, _ICHUNK), jnp.float32),
            pltpu.VMEM((_D, _CCH, _ICHUNK), jnp.float32),
            pltpu.VMEM((_J_PER_W,), jnp.float32),
            pltpu.SemaphoreType.DMA,
        ],
    )
    out = run(tgt_idx, ctx_idx, target_table.T, context_table.T)
    return out.reshape(BATCH, NUM_NS + 1)


# speedup vs baseline: 3.1398x; 3.1398x over previous
import functools

import jax
import jax.numpy as jnp
from jax import lax
from jax.experimental import pallas as pl
from jax.experimental.pallas import tpu as pltpu
from jax.experimental.pallas import tpu_sc as plsc

VOCAB_SIZE = 1000000
EMBEDDING_DIM = 16
NUM_NS = 4
BATCH = 16384

_NC = 2
_NS = 16
_NW = _NC * _NS
_LANES = 16

_B_PER_W = BATCH // _NW
_J_PER_W = _B_PER_W * (NUM_NS + 1)
_CHUNK = 128


def _sc_kernel(tgt_idx_hbm, ctx_idx_hbm, ttab_hbm, ctab_hbm, out_hbm,
               tgt_idx_v, ctx_idx_v, tgt_rows_v, ctx_rows_v, out_v, sem):
    wid = lax.axis_index("s") * _NC + lax.axis_index("c")
    b_base = wid * _B_PER_W
    j_base = wid * _J_PER_W

    pltpu.sync_copy(tgt_idx_hbm.at[pl.ds(b_base, _B_PER_W)], tgt_idx_v)
    pltpu.sync_copy(ctx_idx_hbm.at[pl.ds(j_base, _J_PER_W)], ctx_idx_v)

    copies = []
    for k in range(_B_PER_W // _CHUNK):
        copies.append(pltpu.async_copy(
            ttab_hbm.at[tgt_idx_v.at[pl.ds(k * _CHUNK, _CHUNK)]],
            tgt_rows_v.at[pl.ds(k * _CHUNK, _CHUNK)], sem))
    for k in range(_J_PER_W // _CHUNK):
        copies.append(pltpu.async_copy(
            ctab_hbm.at[ctx_idx_v.at[pl.ds(k * _CHUNK, _CHUNK)]],
            ctx_rows_v.at[pl.ds(k * _CHUNK, _CHUNK)], sem))
    for c in copies:
        c.wait()

    lanes = lax.iota(jnp.int32, _LANES)

    def body(k, carry):
        jvec = lanes + k * _LANES
        bvec = lax.div(jvec, NUM_NS + 1)
        acc = jnp.zeros((_LANES,), jnp.float32)
        for d in range(EMBEDDING_DIM):
            dvec = jnp.full((_LANES,), d, jnp.int32)
            cv = plsc.load_gather(ctx_rows_v, [jvec, dvec])
            tv = plsc.load_gather(tgt_rows_v, [bvec, dvec])
            acc = acc + cv * tv
        out_v[pl.ds(k * _LANES, _LANES)] = acc
        return carry

    lax.fori_loop(0, _J_PER_W // _LANES, body, 0)

    pltpu.sync_copy(out_v, out_hbm.at[pl.ds(j_base, _J_PER_W)])


@jax.jit
def kernel(target, context, target_table, context_table):
    tgt_idx = target.reshape(BATCH)
    ctx_idx = context.reshape(BATCH * (NUM_NS + 1))

    run = pl.kernel(
        _sc_kernel,
        out_type=jax.ShapeDtypeStruct((BATCH * (NUM_NS + 1),), jnp.float32),
        mesh=plsc.VectorSubcoreMesh(core_axis_name="c", subcore_axis_name="s"),
        compiler_params=pltpu.CompilerParams(
            needs_layout_passes=False, use_tc_tiling_on_sc=False),
        scratch_types=[
            pltpu.VMEM((_B_PER_W,), jnp.int32),
            pltpu.VMEM((_J_PER_W,), jnp.int32),
            pltpu.VMEM((_B_PER_W, EMBEDDING_DIM), jnp.float32),
            pltpu.VMEM((_J_PER_W, EMBEDDING_DIM), jnp.float32),
            pltpu.VMEM((_J_PER_W,), jnp.float32),
            pltpu.SemaphoreType.DMA,
        ],
    )
    out = run(tgt_idx, ctx_idx, target_table, context_table)
    return out.reshape(BATCH, NUM_NS + 1)
